# baseline (speedup 1.0000x reference)
import functools

import jax
import jax.numpy as jnp
from jax import lax
from jax.experimental import pallas as pl
from jax.experimental.pallas import tpu as pltpu
from jax.experimental.pallas import tpu_sc as plsc


def kernel(nodes, table):
    (B,) = nodes.shape
    V, D = table.shape
    L = 16

    info = plsc.get_sparse_core_info()
    NC, NS = info.num_cores, info.num_subcores
    NW = NC * NS
    b_per_w = B // NW
    CHUNK = 128
    n_chunks = b_per_w // CHUNK

    V_pad = ((V + 7) // 8) * 8
    P = V_pad // 2

    mesh = plsc.VectorSubcoreMesh(core_axis_name="c", subcore_axis_name="s")

    @functools.partial(
        pl.kernel,
        mesh=mesh,
        out_type=jax.ShapeDtypeStruct((D, B), jnp.float32),
        scratch_types=[
            pltpu.VMEM((b_per_w,), jnp.int32),
            pltpu.VMEM((b_per_w,), jnp.int32),
            pltpu.VMEM((n_chunks, CHUNK, 2 * D), jnp.float32),
            pltpu.VMEM((D, b_per_w), jnp.float32),
            pltpu.SemaphoreType.DMA,
        ],
        compiler_params=pltpu.CompilerParams(
            use_tc_tiling_on_sc=True, needs_layout_passes=False
        ),
    )
    def emb(t128_hbm, idx_hbm, outT_hbm, idx_v, pair_v, rows_v, stage_v, sem):
        wid = lax.axis_index("s") * NC + lax.axis_index("c")
        base = wid * b_per_w
        pltpu.sync_copy(idx_hbm.at[pl.ds(base, b_per_w)], idx_v)

        def mk_pairs(m, _):
            iv = idx_v[pl.ds(m * L, L)]
            pair_v[pl.ds(m * L, L)] = lax.shift_right_logical(iv, 1)
            return _

        lax.fori_loop(0, b_per_w // L, mk_pairs, 0)

        copies = [
            pltpu.async_copy(
                t128_hbm.at[pair_v.at[pl.ds(j * CHUNK, CHUNK)]],
                rows_v.at[j],
                sem,
            )
            for j in range(n_chunks)
        ]
        for c in copies:
            c.wait()

        def extract(m, _):
            jc = m // (CHUNK // L)
            jm = m % (CHUNK // L)
            iv = idx_v[pl.ds(m * L, L)]
            off_vec = (iv & 1) * D
            for l in range(L):
                off = off_vec[l]
                col = m * L + l
                for k in range(D // L):
                    row16 = rows_v[jc, jm * L + l, pl.ds(off + k * L, L)]
                    plsc.store_scatter(
                        stage_v,
                        [k * L + lax.iota(jnp.int32, L),
                         jnp.full((L,), col, jnp.int32)],
                        row16,
                    )
            return _

        lax.fori_loop(0, b_per_w // L, extract, 0)

        pltpu.sync_copy(stage_v, outT_hbm.at[:, pl.ds(base, b_per_w)])

    t128 = jnp.pad(table, ((0, V_pad - V), (0, 0))).reshape(P, 2 * D)
    idx = nodes.astype(jnp.int32)
    outT = emb(t128, idx)
    return outT.T


# speedup vs baseline: 1.4852x; 1.4852x over previous
import functools

import jax
import jax.numpy as jnp
from jax import lax
from jax.experimental import pallas as pl
from jax.experimental.pallas import tpu as pltpu
from jax.experimental.pallas import tpu_sc as plsc


def kernel(nodes, table):
    (B,) = nodes.shape
    V, D = table.shape
    L = 16

    info = plsc.get_sparse_core_info()
    NC, NS = info.num_cores, info.num_subcores
    NW = NC * NS
    b_per_w = B // NW
    CHUNK = 128
    n_chunks = b_per_w // CHUNK

    TR = D // 8
    TC_ALL = B // 128
    tc_per_w = TC_ALL // NW

    mesh = plsc.VectorSubcoreMesh(core_axis_name="c", subcore_axis_name="s")

    @functools.partial(
        pl.kernel,
        mesh=mesh,
        out_type=jax.ShapeDtypeStruct((NW, n_chunks, CHUNK, D), jnp.float32),
        scratch_types=[
            pltpu.VMEM((n_chunks, CHUNK), jnp.int32),
            pltpu.VMEM((n_chunks, CHUNK, D), jnp.float32),
            pltpu.SemaphoreType.DMA,
        ],
        compiler_params=pltpu.CompilerParams(use_tc_tiling_on_sc=False),
    )
    def gather_rows(table_hbm, idx_hbm, out_hbm, idx_v, rows_v, sem):
        wid = lax.axis_index("s") * NC + lax.axis_index("c")
        pltpu.sync_copy(idx_hbm.at[wid], idx_v)
        copies = [
            pltpu.async_copy(table_hbm.at[idx_v.at[j]], rows_v.at[j], sem)
            for j in range(n_chunks)
        ]
        for c in copies:
            c.wait()
        pltpu.sync_copy(rows_v, out_hbm.at[wid])

    @functools.partial(
        pl.kernel,
        mesh=mesh,
        out_type=jax.ShapeDtypeStruct((TR, TC_ALL, 8, 128), jnp.float32),
        scratch_types=[
            pltpu.VMEM((n_chunks, CHUNK, D), jnp.float32),
            pltpu.VMEM((TR, tc_per_w, 8, 128), jnp.float32),
        ],
        compiler_params=pltpu.CompilerParams(
            use_tc_tiling_on_sc=False, needs_layout_passes=False
        ),
    )
    def reformat(rows_hbm, out_hbm, rows_v, stage_v):
        wid = lax.axis_index("s") * NC + lax.axis_index("c")
        pltpu.sync_copy(rows_hbm.at[wid], rows_v)

        lane = lax.iota(jnp.int32, L)
        ge8 = lax.shift_right_logical(lane, 3)
        dsub = lane & 7

        def put(j, _):
            jc = j // CHUNK
            jj = j % CHUNK
            for k in range(D // L):
                row16 = rows_v[jc, jj, pl.ds(k * L, L)]
                plsc.store_scatter(
                    stage_v,
                    [2 * k + ge8,
                     jnp.full((L,), jc, jnp.int32),
                     dsub,
                     jnp.full((L,), jj, jnp.int32)],
                    row16,
                )
            return _

        lax.fori_loop(0, b_per_w, put, 0)
        pltpu.sync_copy(stage_v, out_hbm.at[:, pl.ds(wid * tc_per_w, tc_per_w)])

    idx = nodes.astype(jnp.int32).reshape(NW, n_chunks, CHUNK)
    rows = gather_rows(table, idx)
    out4 = reformat(rows)
    return out4.transpose(1, 3, 0, 2).reshape(B, D)


# speedup vs baseline: 2.3506x; 1.5827x over previous
import functools

import jax
import jax.numpy as jnp
from jax import lax
from jax.experimental import pallas as pl
from jax.experimental.pallas import tpu as pltpu
from jax.experimental.pallas import tpu_sc as plsc


def kernel(nodes, table):
    (B,) = nodes.shape
    V, D = table.shape
    L = 16

    info = plsc.get_sparse_core_info()
    NC, NS = info.num_cores, info.num_subcores
    NW = NC * NS
    b_per_w = B // NW
    CH = 32
    n_ch = b_per_w // CH

    TR = D // 8
    TC_ALL = B // 128
    tc_per_w = TC_ALL // NW

    mesh = plsc.VectorSubcoreMesh(core_axis_name="c", subcore_axis_name="s")

    @functools.partial(
        pl.kernel,
        mesh=mesh,
        out_type=jax.ShapeDtypeStruct((TR, TC_ALL, 8, 128), jnp.float32),
        scratch_types=[
            pltpu.VMEM((b_per_w,), jnp.int32),
            pltpu.VMEM((CH, 8, D), jnp.float32),
            pltpu.VMEM((TR, tc_per_w, 8, 128), jnp.float32),
            pltpu.SemaphoreType.DMA,
        ],
        compiler_params=pltpu.CompilerParams(
            use_tc_tiling_on_sc=True, needs_layout_passes=False
        ),
    )
    def emb(table_hbm, idx_hbm, out_hbm, idx_v, rows_v, stage_v, sem):
        wid = lax.axis_index("s") * NC + lax.axis_index("c")
        base = wid * b_per_w
        pltpu.sync_copy(idx_hbm.at[pl.ds(base, b_per_w)], idx_v)

        lane = lax.iota(jnp.int32, L)
        ge8 = lax.shift_right_logical(lane, 3)
        dsub = lane & 7

        def chunk(c, _):
            for m in range(CH // L):
                iv = idx_v[pl.ds(c * CH + m * L, L)]
                gv = lax.shift_right_logical(iv, 3) * 8
                for l in range(L):
                    g = pl.multiple_of(gv[l], 8)
                    pltpu.async_copy(
                        table_hbm.at[pl.ds(g, 8)],
                        rows_v.at[m * L + l],
                        sem,
                    )
            for _i in range(CH):
                pltpu.make_async_copy(
                    table_hbm.at[pl.ds(0, 8)], rows_v.at[0], sem
                ).wait()

            for m in range(CH // L):
                iv = idx_v[pl.ds(c * CH + m * L, L)]
                sv = iv & 7
                for l in range(L):
                    s = sv[l]
                    j = c * CH + m * L + l
                    jc = j // 128
                    jj = j % 128
                    for k in range(D // L):
                        row16 = rows_v[m * L + l, s, pl.ds(k * L, L)]
                        plsc.store_scatter(
                            stage_v,
                            [2 * k + ge8,
                             jnp.full((L,), jc, jnp.int32),
                             dsub,
                             jnp.full((L,), jj, jnp.int32)],
                            row16,
                        )
            return _

        lax.fori_loop(0, n_ch, chunk, 0)
        pltpu.sync_copy(stage_v, out_hbm.at[:, pl.ds(wid * tc_per_w, tc_per_w)])

    idx = nodes.astype(jnp.int32)
    out4 = emb(table, idx)
    return out4.transpose(1, 3, 0, 2).reshape(B, D)


# speedup vs baseline: 2.4326x; 1.0349x over previous
import functools

import jax
import jax.numpy as jnp
from jax import lax
from jax.experimental import pallas as pl
from jax.experimental.pallas import tpu as pltpu
from jax.experimental.pallas import tpu_sc as plsc


def kernel(nodes, table):
    (B,) = nodes.shape
    V, D = table.shape
    L = 16

    info = plsc.get_sparse_core_info()
    NC, NS = info.num_cores, info.num_subcores
    NW = NC * NS
    b_per_w = B // NW
    CH = 32
    n_ch = b_per_w // CH

    TR = D // 8
    TC_ALL = B // 128
    tc_per_w = TC_ALL // NW

    mesh = plsc.VectorSubcoreMesh(core_axis_name="c", subcore_axis_name="s")

    @functools.partial(
        pl.kernel,
        mesh=mesh,
        out_type=jax.ShapeDtypeStruct((TR, TC_ALL, 8, 128), jnp.float32),
        scratch_types=[
            pltpu.VMEM((b_per_w,), jnp.int32),
            pltpu.VMEM((2, CH, 8, D), jnp.float32),
            pltpu.VMEM((TR, tc_per_w, 8, 128), jnp.float32),
            pltpu.SemaphoreType.DMA,
            pltpu.SemaphoreType.DMA,
        ],
        compiler_params=pltpu.CompilerParams(
            use_tc_tiling_on_sc=True, needs_layout_passes=False
        ),
    )
    def emb(table_hbm, idx_hbm, out_hbm, idx_v, rows_v, stage_v, sem0, sem1):
        wid = lax.axis_index("s") * NC + lax.axis_index("c")
        base = wid * b_per_w
        pltpu.sync_copy(idx_hbm.at[pl.ds(base, b_per_w)], idx_v)

        lane = lax.iota(jnp.int32, L)
        ge8 = lax.shift_right_logical(lane, 3)
        dsub = lane & 7

        def fire(c, buf, sem):
            for m in range(CH // L):
                iv = idx_v[pl.ds(c * CH + m * L, L)]
                gv = lax.shift_right_logical(iv, 3) * 8
                for l in range(L):
                    g = pl.multiple_of(gv[l], 8)
                    pltpu.async_copy(
                        table_hbm.at[pl.ds(g, 8)],
                        rows_v.at[buf, m * L + l],
                        sem,
                    )

        def drain(sem):
            for _i in range(CH):
                pltpu.make_async_copy(
                    table_hbm.at[pl.ds(0, 8)], rows_v.at[0, 0], sem
                ).wait()

        def extract(c, buf):
            for m in range(CH // L):
                iv = idx_v[pl.ds(c * CH + m * L, L)]
                sv = iv & 7
                for l in range(L):
                    s = sv[l]
                    j = c * CH + m * L + l
                    jc = j // 128
                    jj = j % 128
                    for k in range(D // L):
                        row16 = rows_v[buf, m * L + l, s, pl.ds(k * L, L)]
                        plsc.store_scatter(
                            stage_v,
                            [2 * k + ge8,
                             jnp.full((L,), jc, jnp.int32),
                             dsub,
                             jnp.full((L,), jj, jnp.int32)],
                            row16,
                        )

        fire(0, 0, sem0)

        def pair(c2, _):
            c0 = c2 * 2
            fire(c0 + 1, 1, sem1)
            drain(sem0)
            extract(c0, 0)

            @pl.when(c0 + 2 < n_ch)
            def _fire_next():
                fire(c0 + 2, 0, sem0)

            drain(sem1)
            extract(c0 + 1, 1)
            return _

        lax.fori_loop(0, n_ch // 2, pair, 0)
        pltpu.sync_copy(stage_v, out_hbm.at[:, pl.ds(wid * tc_per_w, tc_per_w)])

    idx = nodes.astype(jnp.int32)
    out4 = emb(table, idx)
    return out4.transpose(1, 3, 0, 2).reshape(B, D)


# speedup vs baseline: 3.4691x; 1.4261x over previous
import functools

import jax
import jax.numpy as jnp
from jax import lax
from jax.experimental import pallas as pl
from jax.experimental.pallas import tpu as pltpu
from jax.experimental.pallas import tpu_sc as plsc


def kernel(nodes, table):
    (B,) = nodes.shape
    V, D = table.shape
    L = 16

    info = plsc.get_sparse_core_info()
    NC, NS = info.num_cores, info.num_subcores
    NW = NC * NS
    b_per_w = B // NW
    CH = 32
    n_ch = b_per_w // CH

    TR = D // 8
    TC_ALL = B // 128
    tc_per_w = TC_ALL // NW

    mesh = plsc.VectorSubcoreMesh(core_axis_name="c", subcore_axis_name="s")

    @functools.partial(
        pl.kernel,
        mesh=mesh,
        out_type=jax.ShapeDtypeStruct((TR, TC_ALL, 8, 128), jnp.float32),
        scratch_types=[
            pltpu.VMEM((b_per_w,), jnp.int32),
            pltpu.VMEM((2, CH, 8, D), jnp.float32),
            pltpu.VMEM((TR, tc_per_w, 8, 128), jnp.float32),
            pltpu.SemaphoreType.DMA,
            pltpu.SemaphoreType.DMA,
        ],
        compiler_params=pltpu.CompilerParams(
            use_tc_tiling_on_sc=True, needs_layout_passes=False
        ),
    )
    def emb(table_hbm, idx_hbm, out_hbm, idx_v, rows_v, stage_v, sem0, sem1):
        wid = lax.axis_index("s") * NC + lax.axis_index("c")
        base = wid * b_per_w
        pltpu.sync_copy(idx_hbm.at[pl.ds(base, b_per_w)], idx_v)

        lane = lax.iota(jnp.int32, L)
        ge8 = lax.shift_right_logical(lane, 3)
        dsub = lane & 7

        def fire(c, buf, sem):
            for m in range(CH // L):
                iv = idx_v[pl.ds(c * CH + m * L, L)]
                gv = lax.shift_right_logical(iv, 3) * 8
                for l in range(L):
                    g = pl.multiple_of(gv[l], 8)
                    pltpu.async_copy(
                        table_hbm.at[pl.ds(g, 8)],
                        rows_v.at[buf, m * L + l],
                        sem,
                    )

        def drain(sem):
            for _i in range(CH):
                pltpu.make_async_copy(
                    table_hbm.at[pl.ds(0, 8)], rows_v.at[0, 0], sem
                ).wait()

        def extract(c, buf):
            for m in range(CH // L):
                iv = idx_v[pl.ds(c * CH + m * L, L)]
                sv = iv & 7
                for l in range(L):
                    s = sv[l]
                    j = c * CH + m * L + l
                    jc = j // 128
                    jj = j % 128
                    for k in range(D // L):
                        row16 = rows_v[buf, m * L + l, s, pl.ds(k * L, L)]
                        plsc.store_scatter(
                            stage_v,
                            [2 * k + ge8,
                             jnp.full((L,), jc, jnp.int32),
                             dsub,
                             jnp.full((L,), jj, jnp.int32)],
                            row16,
                        )

        fire(0, 0, sem0)

        def pair(c2, _):
            c0 = c2 * 2
            fire(c0 + 1, 1, sem1)
            drain(sem0)
            extract(c0, 0)

            @pl.when(c0 + 2 < n_ch)
            def _fire_next():
                fire(c0 + 2, 0, sem0)

            drain(sem1)
            extract(c0 + 1, 1)
            return _

        lax.fori_loop(0, n_ch // 2, pair, 0)
        pltpu.sync_copy(stage_v, out_hbm.at[:, pl.ds(wid * tc_per_w, tc_per_w)])

    idx = nodes.astype(jnp.int32)
    t_rm = table.at[0, 0].set(table[0, 0])
    out4 = emb(t_rm, idx)
    return out4.transpose(1, 3, 0, 2).reshape(B, D)


# speedup vs baseline: 3.4893x; 1.0058x over previous
import functools

import jax
import jax.numpy as jnp
from jax import lax
from jax.experimental import pallas as pl
from jax.experimental.pallas import tpu as pltpu
from jax.experimental.pallas import tpu_sc as plsc


def kernel(nodes, table):
    (B,) = nodes.shape
    V, D = table.shape
    L = 16

    info = plsc.get_sparse_core_info()
    NC, NS = info.num_cores, info.num_subcores
    NW = NC * NS
    b_per_w = B // NW
    CH = 32
    n_ch = b_per_w // CH

    TR = D // 8
    TC_ALL = B // 128
    tc_per_w = TC_ALL // NW

    mesh = plsc.VectorSubcoreMesh(core_axis_name="c", subcore_axis_name="s")

    @functools.partial(
        pl.kernel,
        mesh=mesh,
        out_type=jax.ShapeDtypeStruct((TR, TC_ALL, 8, 128), jnp.float32),
        scratch_types=[
            pltpu.VMEM((b_per_w,), jnp.int32),
            pltpu.VMEM((2, CH, 8, D), jnp.float32),
            pltpu.VMEM((TR, tc_per_w, 8, 128), jnp.float32),
            pltpu.SemaphoreType.DMA,
            pltpu.SemaphoreType.DMA,
        ],
        compiler_params=pltpu.CompilerParams(
            use_tc_tiling_on_sc=True, needs_layout_passes=False
        ),
    )
    def emb(table_hbm, idx_hbm, out_hbm, idx_v, rows_v, stage_v, sem0, sem1):
        wid = lax.axis_index("s") * NC + lax.axis_index("c")
        base = wid * b_per_w
        pltpu.sync_copy(idx_hbm.at[pl.ds(base, b_per_w)], idx_v)

        lane = lax.iota(jnp.int32, L)
        ge8 = lax.shift_right_logical(lane, 3)
        dsub = lane & 7

        def fire(c, buf, sem):
            for m in range(CH // L):
                iv = idx_v[pl.ds(c * CH + m * L, L)]
                gv = lax.shift_right_logical(iv, 3) * 8
                for l in range(L):
                    g = pl.multiple_of(gv[l], 8)
                    pltpu.async_copy(
                        table_hbm.at[pl.ds(g, 8)],
                        rows_v.at[buf, m * L + l],
                        sem,
                    )

        def drain(buf, sem):
            pltpu.make_async_copy(
                table_hbm.at[pl.ds(0, CH * 8)],
                rows_v.at[buf].reshape(CH * 8, D),
                sem,
            ).wait()

        def extract(c, buf):
            for m in range(CH // L):
                iv = idx_v[pl.ds(c * CH + m * L, L)]
                sv = iv & 7
                for l in range(L):
                    s = sv[l]
                    j = c * CH + m * L + l
                    jc = j // 128
                    jj = j % 128
                    for k in range(D // L):
                        row16 = rows_v[buf, m * L + l, s, pl.ds(k * L, L)]
                        plsc.store_scatter(
                            stage_v,
                            [2 * k + ge8,
                             jnp.full((L,), jc, jnp.int32),
                             dsub,
                             jnp.full((L,), jj, jnp.int32)],
                            row16,
                        )

        fire(0, 0, sem0)

        def pair(c2, _):
            c0 = c2 * 2
            fire(c0 + 1, 1, sem1)
            drain(0, sem0)
            extract(c0, 0)

            @pl.when(c0 + 2 < n_ch)
            def _fire_next():
                fire(c0 + 2, 0, sem0)

            drain(1, sem1)
            extract(c0 + 1, 1)
            return _

        lax.fori_loop(0, n_ch // 2, pair, 0)
        pltpu.sync_copy(stage_v, out_hbm.at[:, pl.ds(wid * tc_per_w, tc_per_w)])

    idx = nodes.astype(jnp.int32)
    t_rm = table.at[0, 0].set(table[0, 0])
    out4 = emb(t_rm, idx)
    return out4.transpose(1, 3, 0, 2).reshape(B, D)
